# Initial kernel scaffold; baseline (speedup 1.0000x reference)
import jax
import jax.numpy as jnp
from jax.experimental import pallas as pl


def kernel(coeffs, coeffs_derivs, central_atom_index, neigh_atom_index, weight1, bias1):
    raise NotImplementedError("write your pallas kernel here")


# speedup vs baseline: 3.5379x; 3.5379x over previous
import functools

import jax
import jax.numpy as jnp
from jax import lax
from jax.experimental import pallas as pl
from jax.experimental.pallas import tpu as pltpu
from jax.experimental.pallas import tpu_sc as plsc

_NC = 2
_NS = 16
_NW = _NC * _NS
_APAD = 10240


def _f_body(cd_ref, w_ref, f_ref):
    f_ref[...] = -jax.lax.dot_general(
        cd_ref[...], w_ref[0],
        dimension_numbers=(((2,), (0,)), ((), ())),
        preferred_element_type=jnp.float32,
    )


def _compute_f(cd2, w):
    _, m, d = cd2.shape
    bm = 2560
    return pl.pallas_call(
        _f_body,
        grid=(m // bm,),
        in_specs=[
            pl.BlockSpec((3, bm, d), lambda i: (0, i, 0)),
            pl.BlockSpec((1, d), lambda i: (0, 0)),
        ],
        out_specs=pl.BlockSpec((3, bm), lambda i: (0, i)),
        out_shape=jax.ShapeDtypeStruct((3, m), jnp.float32),
    )(cd2, w)


def _sc_segment_sum(f_flat, neigh, m):
    chunk = m // _NW
    sl = (3 * _APAD) // _NS
    mesh = plsc.VectorSubcoreMesh(core_axis_name="c", subcore_axis_name="s")

    @functools.partial(
        pl.kernel,
        out_type=jax.ShapeDtypeStruct((_NC, 3 * _APAD), jnp.float32),
        mesh=mesh,
        scratch_types=[
            pltpu.VMEM((chunk,), jnp.int32),
            pltpu.VMEM((3 * chunk,), jnp.float32),
            pltpu.VMEM((3 * _APAD,), jnp.float32),
            pltpu.VMEM_SHARED((_NS, 3 * _APAD), jnp.float32),
            pltpu.VMEM((sl,), jnp.float32),
            pltpu.VMEM((sl,), jnp.float32),
        ],
        compiler_params=pltpu.CompilerParams(needs_layout_passes=False),
    )
    def segsum(f_hbm, neigh_hbm, out_hbm, idx_v, vals_v, acc_v, shared, red_v, tmp_v):
        cid = lax.axis_index("c")
        sid = lax.axis_index("s")
        wid = cid * _NS + sid
        base = wid * chunk

        pltpu.sync_copy(neigh_hbm.at[pl.ds(base, chunk)], idx_v)
        for kk in range(3):
            pltpu.sync_copy(
                f_hbm.at[pl.ds(kk * m + base, chunk)],
                vals_v.at[pl.ds(kk * chunk, chunk)],
            )

        zero = jnp.zeros((16,), jnp.float32)

        def zbody(i, c):
            acc_v[pl.ds(i * 16, 16)] = zero
            return c

        lax.fori_loop(0, (3 * _APAD) // 16, zbody, 0)

        def sbody(j, c):
            idx = idx_v[pl.ds(j * 16, 16)]
            for kk in range(3):
                v = vals_v[pl.ds(kk * chunk + j * 16, 16)]
                plsc.addupdate_scatter(acc_v, [idx + kk * _APAD], v)
            return c

        lax.fori_loop(0, chunk // 16, sbody, 0)

        pltpu.sync_copy(acc_v, shared.at[sid])
        plsc.subcore_barrier()

        rbase = sid * sl
        pltpu.sync_copy(shared.at[0, pl.ds(rbase, sl)], red_v)
        for s in range(1, _NS):
            pltpu.sync_copy(shared.at[s, pl.ds(rbase, sl)], tmp_v)

            def abody(i, c):
                red_v[pl.ds(i * 16, 16)] = (
                    red_v[pl.ds(i * 16, 16)] + tmp_v[pl.ds(i * 16, 16)]
                )
                return c

            lax.fori_loop(0, sl // 16, abody, 0)
        pltpu.sync_copy(red_v, out_hbm.at[cid, pl.ds(rbase, sl)])

    return segsum(f_flat, neigh)


def _merge_body(c_ref, w_ref, b_ref, parts_ref, e_ref, f_ref):
    n_atoms = c_ref.shape[0]
    s = jnp.sum(c_ref[...] * w_ref[0][None, :])
    e_ref[0, 0] = s / n_atoms + b_ref[0]
    f_ref[...] = parts_ref[0] + parts_ref[1]


def _merge(coeffs2, w, b, parts):
    n, d = coeffs2.shape
    return pl.pallas_call(
        _merge_body,
        in_specs=[
            pl.BlockSpec((n, d), lambda: (0, 0)),
            pl.BlockSpec((1, d), lambda: (0, 0)),
            pl.BlockSpec(memory_space=pltpu.SMEM),
            pl.BlockSpec((_NC, 3, _APAD), lambda: (0, 0, 0)),
        ],
        out_specs=[
            pl.BlockSpec(memory_space=pltpu.SMEM),
            pl.BlockSpec((3, _APAD), lambda: (0, 0)),
        ],
        out_shape=[
            jax.ShapeDtypeStruct((1, 1), jnp.float32),
            jax.ShapeDtypeStruct((3, _APAD), jnp.float32),
        ],
    )(coeffs2, w, b, parts)


def kernel(coeffs, coeffs_derivs, central_atom_index, neigh_atom_index, weight1, bias1):
    del central_atom_index
    num_atoms = coeffs.shape[1]
    m = coeffs_derivs.shape[2]

    cd2 = coeffs_derivs.reshape(3, m, coeffs_derivs.shape[3])
    f = _compute_f(cd2, weight1)
    parts = _sc_segment_sum(f.reshape(3 * m), neigh_atom_index, m)
    e, fmerged = _merge(
        coeffs.reshape(num_atoms, -1), weight1, bias1,
        parts.reshape(_NC, 3, _APAD),
    )
    e_pa = e.reshape(1)
    out_f = fmerged[:, :num_atoms][None]
    return (e_pa, out_f)


# speedup vs baseline: 3.6319x; 1.0266x over previous
import functools

import jax
import jax.numpy as jnp
from jax import lax
from jax.experimental import pallas as pl
from jax.experimental.pallas import tpu as pltpu
from jax.experimental.pallas import tpu_sc as plsc

_NC = 2
_NS = 16
_NW = _NC * _NS
_APAD = 10240


def _f_body(cd_ref, w_ref, f_ref):
    f_ref[...] = -jax.lax.dot_general(
        cd_ref[...], w_ref[0],
        dimension_numbers=(((2,), (0,)), ((), ())),
        preferred_element_type=jnp.float32,
    )


def _compute_f(cd2, w):
    _, m, d = cd2.shape
    bm = 6400
    return pl.pallas_call(
        _f_body,
        grid=(m // bm,),
        in_specs=[
            pl.BlockSpec((3, bm, d), lambda i: (0, i, 0)),
            pl.BlockSpec((1, d), lambda i: (0, 0)),
        ],
        out_specs=pl.BlockSpec((3, bm), lambda i: (0, i)),
        out_shape=jax.ShapeDtypeStruct((3, m), jnp.float32),
    )(cd2, w)


def _sc_segment_sum(f_flat, neigh, m):
    chunk = m // _NW
    sl = (3 * _APAD) // _NS
    mesh = plsc.VectorSubcoreMesh(core_axis_name="c", subcore_axis_name="s")

    @functools.partial(
        pl.kernel,
        out_type=jax.ShapeDtypeStruct((_NC, 3 * _APAD), jnp.float32),
        mesh=mesh,
        scratch_types=[
            pltpu.VMEM((chunk,), jnp.int32),
            pltpu.VMEM((3 * chunk,), jnp.float32),
            pltpu.VMEM((3 * _APAD,), jnp.float32),
            pltpu.VMEM_SHARED((_NS, 3 * _APAD), jnp.float32),
            pltpu.VMEM((sl,), jnp.float32),
            pltpu.VMEM((sl,), jnp.float32),
        ],
        compiler_params=pltpu.CompilerParams(needs_layout_passes=False),
    )
    def segsum(f_hbm, neigh_hbm, out_hbm, idx_v, vals_v, acc_v, shared, red_v, tmp_v):
        cid = lax.axis_index("c")
        sid = lax.axis_index("s")
        wid = cid * _NS + sid
        base = wid * chunk

        pltpu.sync_copy(neigh_hbm.at[pl.ds(base, chunk)], idx_v)
        for kk in range(3):
            pltpu.sync_copy(
                f_hbm.at[pl.ds(kk * m + base, chunk)],
                vals_v.at[pl.ds(kk * chunk, chunk)],
            )

        zero = jnp.zeros((16,), jnp.float32)

        def zbody(i, c):
            acc_v[pl.ds(i * 16, 16)] = zero
            return c

        lax.fori_loop(0, (3 * _APAD) // 16, zbody, 0)

        def sbody(j, c):
            idx = idx_v[pl.ds(j * 16, 16)]
            for kk in range(3):
                v = vals_v[pl.ds(kk * chunk + j * 16, 16)]
                plsc.addupdate_scatter(acc_v, [idx + kk * _APAD], v)
            return c

        lax.fori_loop(0, chunk // 16, sbody, 0)

        pltpu.sync_copy(acc_v, shared.at[sid])
        plsc.subcore_barrier()

        rbase = sid * sl
        pltpu.sync_copy(shared.at[0, pl.ds(rbase, sl)], red_v)
        for s in range(1, _NS):
            pltpu.sync_copy(shared.at[s, pl.ds(rbase, sl)], tmp_v)

            def abody(i, c):
                red_v[pl.ds(i * 16, 16)] = (
                    red_v[pl.ds(i * 16, 16)] + tmp_v[pl.ds(i * 16, 16)]
                )
                return c

            lax.fori_loop(0, sl // 16, abody, 0)
        pltpu.sync_copy(red_v, out_hbm.at[cid, pl.ds(rbase, sl)])

    return segsum(f_flat, neigh)


def _merge_body(c_ref, w_ref, b_ref, parts_ref, e_ref, f_ref):
    n_atoms = c_ref.shape[0]
    s = jnp.sum(c_ref[...] * w_ref[0][None, :])
    e_ref[0, 0] = s / n_atoms + b_ref[0]
    f_ref[...] = parts_ref[0] + parts_ref[1]


def _merge(coeffs2, w, b, parts):
    n, d = coeffs2.shape
    return pl.pallas_call(
        _merge_body,
        in_specs=[
            pl.BlockSpec((n, d), lambda: (0, 0)),
            pl.BlockSpec((1, d), lambda: (0, 0)),
            pl.BlockSpec(memory_space=pltpu.SMEM),
            pl.BlockSpec((_NC, 3, _APAD), lambda: (0, 0, 0)),
        ],
        out_specs=[
            pl.BlockSpec(memory_space=pltpu.SMEM),
            pl.BlockSpec((3, _APAD), lambda: (0, 0)),
        ],
        out_shape=[
            jax.ShapeDtypeStruct((1, 1), jnp.float32),
            jax.ShapeDtypeStruct((3, _APAD), jnp.float32),
        ],
    )(coeffs2, w, b, parts)


def kernel(coeffs, coeffs_derivs, central_atom_index, neigh_atom_index, weight1, bias1):
    del central_atom_index
    num_atoms = coeffs.shape[1]
    m = coeffs_derivs.shape[2]

    cd2 = coeffs_derivs.reshape(3, m, coeffs_derivs.shape[3])
    f = _compute_f(cd2, weight1)
    parts = _sc_segment_sum(f.reshape(3 * m), neigh_atom_index, m)
    e, fmerged = _merge(
        coeffs.reshape(num_atoms, -1), weight1, bias1,
        parts.reshape(_NC, 3, _APAD),
    )
    e_pa = e.reshape(1)
    out_f = fmerged[:, :num_atoms][None]
    return (e_pa, out_f)
